# baseline (speedup 1.0000x reference)
import functools

import jax
import jax.numpy as jnp
from jax import lax
from jax.experimental import pallas as pl
from jax.experimental.pallas import tpu as pltpu
from jax.experimental.pallas import tpu_sc as plsc

_B = 16384
_DIM = 64
_NC = 2
_NS = 16
_LANES = 16
_NW = _NC * _NS
_BPW = _B // _NW
_RND = 128
_NRND = _BPW // _RND


def _permute16(x, idx):
    dn = lax.GatherDimensionNumbers(
        offset_dims=(), collapsed_slice_dims=(0,), start_index_map=(0,))
    return lax.gather(x, idx[:, None], dn, (1,),
                      mode=lax.GatherScatterMode.PROMISE_IN_BOUNDS)


def _sqrt16(x):
    y = jnp.where(x > 4096.0, 0.001 * x + 64.0, 0.0625 * x + 4.0)
    for _ in range(7):
        y = 0.5 * (y + x / y)
    return jnp.where(x > 0.0, y, 0.0)


def _body(heads_hbm, rels_hbm, tails_hbm, ent_hbm, relt_hbm, out_hbm,
          hidx, ridx, tidx, hpar, rpar, tpar,
          h_blk, r_blk, t_blk, scores_v, sem):
    wid = lax.axis_index("s") * _NC + lax.axis_index("c")
    base = wid * _BPW

    pltpu.sync_copy(heads_hbm.at[pl.ds(base, _BPW)], hidx)
    pltpu.sync_copy(rels_hbm.at[pl.ds(base, _BPW)], ridx)
    pltpu.sync_copy(tails_hbm.at[pl.ds(base, _BPW)], tidx)

    def split_body(v, carry):
        sl = pl.ds(v * _LANES, _LANES)
        for idxv, parv in ((hidx, hpar), (ridx, rpar), (tidx, tpar)):
            x = idxv[sl]
            parv[sl] = lax.bitwise_and(x, 1)
            idxv[sl] = lax.shift_right_logical(x, 1)
        return carry

    lax.fori_loop(0, _BPW // _LANES, split_body, 0)

    lanes = jnp.arange(_LANES, dtype=jnp.int32)
    perms = [lanes ^ (1 << p) for p in range(4)]

    for rnd in range(_NRND):
        sl128 = pl.ds(rnd * _RND, _RND)
        handles = [
            pltpu.async_copy(ent_hbm.at[hidx.at[sl128]], h_blk, sem),
            pltpu.async_copy(relt_hbm.at[ridx.at[sl128]], r_blk, sem),
            pltpu.async_copy(ent_hbm.at[tidx.at[sl128]], t_blk, sem),
        ]
        for h in handles:
            h.wait()

        def grp_body(g, carry):
            res = jnp.zeros((_LANES,), jnp.float32)
            gsl = pl.ds(rnd * _RND + g * _LANES, _LANES)
            phv = hpar[gsl].astype(jnp.float32)
            prv = rpar[gsl].astype(jnp.float32)
            ptv = tpar[gsl].astype(jnp.float32)
            for l in range(_LANES):
                i = g * _LANES + l
                ph = phv[l]
                pr = prv[l]
                pt = ptv[l]
                acc = jnp.zeros((_LANES,), jnp.float32)
                for j in range(_DIM // _LANES):
                    lo = pl.ds(j * _LANES, _LANES)
                    hi = pl.ds(_DIM + j * _LANES, _LANES)
                    hv = h_blk[i, lo] + ph * (h_blk[i, hi] - h_blk[i, lo])
                    rv = r_blk[i, lo] + pr * (r_blk[i, hi] - r_blk[i, lo])
                    tv = t_blk[i, lo] + pt * (t_blk[i, hi] - t_blk[i, lo])
                    d = (hv + rv) - tv
                    acc = acc + d * d
                for p in perms:
                    acc = acc + _permute16(acc, p)
                res = jnp.where(lanes == l, acc, res)
            scores_v[pl.ds(rnd * _RND + g * _LANES, _LANES)] = _sqrt16(res)
            return carry

        lax.fori_loop(0, _RND // _LANES, grp_body, 0)

    pltpu.sync_copy(scores_v, out_hbm.at[pl.ds(base, _BPW)])


_mesh = plsc.VectorSubcoreMesh(core_axis_name="c", subcore_axis_name="s")

_kernel_call = pl.kernel(
    _body,
    out_type=jax.ShapeDtypeStruct((_B,), jnp.float32),
    scratch_types=[
        pltpu.VMEM((_BPW,), jnp.int32),
        pltpu.VMEM((_BPW,), jnp.int32),
        pltpu.VMEM((_BPW,), jnp.int32),
        pltpu.VMEM((_BPW,), jnp.int32),
        pltpu.VMEM((_BPW,), jnp.int32),
        pltpu.VMEM((_BPW,), jnp.int32),
        pltpu.VMEM((_RND, 2 * _DIM), jnp.float32),
        pltpu.VMEM((_RND, 2 * _DIM), jnp.float32),
        pltpu.VMEM((_RND, 2 * _DIM), jnp.float32),
        pltpu.VMEM((_BPW,), jnp.float32),
        pltpu.SemaphoreType.DMA,
    ],
    mesh=_mesh,
    compiler_params=pltpu.CompilerParams(use_tc_tiling_on_sc=True),
)


@jax.jit
def kernel(heads, rels, tails, entity_table, rel_table):
    ent2 = entity_table.reshape(entity_table.shape[0] // 2, 2 * _DIM)
    rel2 = rel_table.reshape(rel_table.shape[0] // 2, 2 * _DIM)
    return _kernel_call(heads, rels, tails, ent2, rel2)


# speedup vs baseline: 1.0107x; 1.0107x over previous
import jax
import jax.numpy as jnp
from jax import lax
from jax.experimental import pallas as pl
from jax.experimental.pallas import tpu as pltpu
from jax.experimental.pallas import tpu_sc as plsc

_B = 16384
_DIM = 64
_NC = 2
_NS = 16
_LANES = 16
_NW = _NC * _NS
_BPW = _B // _NW
_RND = 128
_NRND = _BPW // _RND


def _permute16(x, idx):
    dn = lax.GatherDimensionNumbers(
        offset_dims=(), collapsed_slice_dims=(0,), start_index_map=(0,))
    return lax.gather(x, idx[:, None], dn, (1,),
                      mode=lax.GatherScatterMode.PROMISE_IN_BOUNDS)


def _sqrt16(x):
    y = jnp.where(x > 4096.0, 0.001 * x + 64.0, 0.0625 * x + 4.0)
    for _ in range(7):
        y = 0.5 * (y + x / y)
    return jnp.where(x > 0.0, y, 0.0)


def _body(heads_hbm, rels_hbm, tails_hbm, ent_hbm, relt_hbm, out_hbm,
          hidx, ridx, tidx, hpar, rpar, tpar,
          h0, r0, t0, h1, r1, t1, scores_v, sem0, sem1):
    wid = lax.axis_index("s") * _NC + lax.axis_index("c")
    base = wid * _BPW

    pltpu.sync_copy(heads_hbm.at[pl.ds(base, _BPW)], hidx)
    pltpu.sync_copy(rels_hbm.at[pl.ds(base, _BPW)], ridx)
    pltpu.sync_copy(tails_hbm.at[pl.ds(base, _BPW)], tidx)

    def split_body(v, carry):
        sl = pl.ds(v * _LANES, _LANES)
        for idxv, parv in ((hidx, hpar), (ridx, rpar), (tidx, tpar)):
            x = idxv[sl]
            parv[sl] = lax.bitwise_and(x, 1).astype(jnp.float32)
            idxv[sl] = lax.shift_right_logical(x, 1)
        return carry

    lax.fori_loop(0, _BPW // _LANES, split_body, 0)

    bufs = [(h0, r0, t0, sem0), (h1, r1, t1, sem1)]

    def fire(rnd):
        sl = pl.ds(rnd * _RND, _RND)
        hb, rb, tb, sem = bufs[rnd % 2]
        return [
            pltpu.async_copy(ent_hbm.at[hidx.at[sl]], hb, sem),
            pltpu.async_copy(relt_hbm.at[ridx.at[sl]], rb, sem),
            pltpu.async_copy(ent_hbm.at[tidx.at[sl]], tb, sem),
        ]

    lanes = jnp.arange(_LANES, dtype=jnp.int32)
    handles = fire(0)
    for rnd in range(_NRND):
        for h in handles:
            h.wait()
        hb, rb, tb, _ = bufs[rnd % 2]
        if rnd + 1 < _NRND:
            handles = fire(rnd + 1)

        def grp_body(g, carry):
            gsl = pl.ds(rnd * _RND + g * _LANES, _LANES)
            phv = hpar[gsl]
            prv = rpar[gsl]
            ptv = tpar[gsl]
            vecs = []
            for l in range(_LANES):
                i = g * _LANES + l
                ph = phv[l]
                pr = prv[l]
                pt = ptv[l]
                acc = jnp.zeros((_LANES,), jnp.float32)
                for j in range(_DIM // _LANES):
                    lo = pl.ds(j * _LANES, _LANES)
                    hi = pl.ds(_DIM + j * _LANES, _LANES)
                    hv = hb[i, lo] + ph * (hb[i, hi] - hb[i, lo])
                    rv = rb[i, lo] + pr * (rb[i, hi] - rb[i, lo])
                    tv = tb[i, lo] + pt * (tb[i, hi] - tb[i, lo])
                    d = (hv + rv) - tv
                    acc = acc + d * d
                vecs.append(acc)
            for s in range(4):
                step = 1 << s
                folded = [v + _permute16(v, lanes ^ step) for v in vecs]
                bit = lax.bitwise_and(lax.shift_right_logical(lanes, s), 1)
                vecs = [jnp.where(bit == 0, folded[2 * k], folded[2 * k + 1])
                        for k in range(len(folded) // 2)]
            scores_v[gsl] = _sqrt16(vecs[0])
            return carry

        lax.fori_loop(0, _RND // _LANES, grp_body, 0)

    pltpu.sync_copy(scores_v, out_hbm.at[pl.ds(base, _BPW)])


_mesh = plsc.VectorSubcoreMesh(core_axis_name="c", subcore_axis_name="s")

_kernel_call = pl.kernel(
    _body,
    out_type=jax.ShapeDtypeStruct((_B,), jnp.float32),
    scratch_types=[
        pltpu.VMEM((_BPW,), jnp.int32),
        pltpu.VMEM((_BPW,), jnp.int32),
        pltpu.VMEM((_BPW,), jnp.int32),
        pltpu.VMEM((_BPW,), jnp.float32),
        pltpu.VMEM((_BPW,), jnp.float32),
        pltpu.VMEM((_BPW,), jnp.float32),
        pltpu.VMEM((_RND, 2 * _DIM), jnp.float32),
        pltpu.VMEM((_RND, 2 * _DIM), jnp.float32),
        pltpu.VMEM((_RND, 2 * _DIM), jnp.float32),
        pltpu.VMEM((_RND, 2 * _DIM), jnp.float32),
        pltpu.VMEM((_RND, 2 * _DIM), jnp.float32),
        pltpu.VMEM((_RND, 2 * _DIM), jnp.float32),
        pltpu.VMEM((_BPW,), jnp.float32),
        pltpu.SemaphoreType.DMA,
        pltpu.SemaphoreType.DMA,
    ],
    mesh=_mesh,
    compiler_params=pltpu.CompilerParams(use_tc_tiling_on_sc=True),
)


@jax.jit
def kernel(heads, rels, tails, entity_table, rel_table):
    ent2 = entity_table.reshape(entity_table.shape[0] // 2, 2 * _DIM)
    rel2 = rel_table.reshape(rel_table.shape[0] // 2, 2 * _DIM)
    return _kernel_call(heads, rels, tails, ent2, rel2)
